# Initial kernel scaffold; baseline (speedup 1.0000x reference)
import jax
import jax.numpy as jnp
from jax.experimental import pallas as pl


def kernel(token_ids, embedding_matrix):
    raise NotImplementedError("write your pallas kernel here")


# speedup vs baseline: 1.7980x; 1.7980x over previous
import functools

import jax
import jax.numpy as jnp
from jax import lax
from jax.experimental import pallas as pl
from jax.experimental.pallas import tpu as pltpu
from jax.experimental.pallas import tpu_sc as plsc

BATCH = 16384
HIST_LEN = 50
EMBEDDING_DIM = 64
TOTAL = BATCH * HIST_LEN

_INFO = plsc.get_sparse_core_info()
NUM_CORES = _INFO.num_cores
NUM_SUBCORES = _INFO.num_subcores
NUM_WORKERS = NUM_CORES * NUM_SUBCORES

PER_WORKER = TOTAL // NUM_WORKERS
CHUNK = 512
STEPS = PER_WORKER // CHUNK


def _gather_kernel(table_hbm, idx_hbm, out_hbm, idx_v, rows_v, sem):
    wid = lax.axis_index("s") * NUM_CORES + lax.axis_index("c")
    base = wid * PER_WORKER

    @pl.loop(0, STEPS)
    def _step(i):
        off = base + i * CHUNK
        pltpu.sync_copy(idx_hbm.at[pl.ds(off, CHUNK)], idx_v)
        pltpu.async_copy(table_hbm.at[idx_v], rows_v, sem).wait()
        pltpu.sync_copy(rows_v, out_hbm.at[pl.ds(off, CHUNK)])


@jax.jit
def _gather(table, idx_flat):
    mesh = plsc.VectorSubcoreMesh(core_axis_name="c", subcore_axis_name="s")
    run = functools.partial(
        pl.kernel,
        mesh=mesh,
        out_type=jax.ShapeDtypeStruct((TOTAL, EMBEDDING_DIM), jnp.float32),
        scratch_types=[
            pltpu.VMEM((CHUNK,), jnp.int32),
            pltpu.VMEM((CHUNK, EMBEDDING_DIM), jnp.float32),
            pltpu.SemaphoreType.DMA,
        ],
        compiler_params=pltpu.CompilerParams(use_tc_tiling_on_sc=False),
    )(_gather_kernel)
    return run(table, idx_flat)


def kernel(token_ids, embedding_matrix):
    idx_flat = token_ids.reshape(TOTAL).astype(jnp.int32)
    out = _gather(embedding_matrix, idx_flat)
    return out.reshape(BATCH, HIST_LEN, EMBEDDING_DIM)


# speedup vs baseline: 1.8753x; 1.0430x over previous
import functools

import jax
import jax.numpy as jnp
from jax import lax
from jax.experimental import pallas as pl
from jax.experimental.pallas import tpu as pltpu
from jax.experimental.pallas import tpu_sc as plsc

BATCH = 16384
HIST_LEN = 50
EMBEDDING_DIM = 64
TOTAL = BATCH * HIST_LEN

_INFO = plsc.get_sparse_core_info()
NUM_CORES = _INFO.num_cores
NUM_SUBCORES = _INFO.num_subcores
NUM_WORKERS = NUM_CORES * NUM_SUBCORES

PER_WORKER = TOTAL // NUM_WORKERS
CHUNK = 256
STEPS = PER_WORKER // CHUNK
NBUF = 4


def _gather_kernel(table_hbm, idx_hbm, out_hbm, idx_all, rows, *sems):
    gsems = sems[:NBUF]
    ssems = sems[NBUF:]
    wid = lax.axis_index("s") * NUM_CORES + lax.axis_index("c")
    base = wid * PER_WORKER

    def idx_slice(i):
        return idx_all.at[pl.ds(i * CHUNK, CHUNK)]

    def start_gather(i, b):
        pltpu.async_copy(table_hbm.at[idx_slice(i)], rows.at[b], gsems[b])

    def wait_gather(i, b):
        pltpu.make_async_copy(
            table_hbm.at[idx_slice(i)], rows.at[b], gsems[b]
        ).wait()

    def out_slice(i):
        return out_hbm.at[pl.ds(base + i * CHUNK, CHUNK)]

    def start_store(i, b):
        pltpu.async_copy(rows.at[b], out_slice(i), ssems[b])

    def wait_store(i, b):
        pltpu.make_async_copy(rows.at[b], out_slice(i), ssems[b]).wait()

    pltpu.sync_copy(idx_hbm.at[pl.ds(base, PER_WORKER)], idx_all)

    for b in range(NBUF):
        start_gather(b, b)

    @pl.loop(0, STEPS, step=NBUF)
    def _outer(g):
        for b in range(NBUF):
            i = g + b
            wait_gather(i, b)
            start_store(i, b)

            @pl.when(i + NBUF < STEPS)
            def _():
                wait_store(i, b)
                start_gather(i + NBUF, b)

    for b in range(NBUF):
        wait_store(STEPS - NBUF + b, b)


@jax.jit
def _gather(table, idx_flat):
    mesh = plsc.VectorSubcoreMesh(core_axis_name="c", subcore_axis_name="s")
    run = functools.partial(
        pl.kernel,
        mesh=mesh,
        out_type=jax.ShapeDtypeStruct((TOTAL, EMBEDDING_DIM), jnp.float32),
        scratch_types=[
            pltpu.VMEM((PER_WORKER,), jnp.int32),
            pltpu.VMEM((NBUF, CHUNK, EMBEDDING_DIM), jnp.float32),
        ]
        + [pltpu.SemaphoreType.DMA] * (2 * NBUF),
        compiler_params=pltpu.CompilerParams(use_tc_tiling_on_sc=False),
    )(_gather_kernel)
    return run(table, idx_flat)


def kernel(token_ids, embedding_matrix):
    idx_flat = token_ids.reshape(TOTAL).astype(jnp.int32)
    out = _gather(embedding_matrix, idx_flat)
    return out.reshape(BATCH, HIST_LEN, EMBEDDING_DIM)


# speedup vs baseline: 1.8756x; 1.0001x over previous
import functools

import jax
import jax.numpy as jnp
from jax import lax
from jax.experimental import pallas as pl
from jax.experimental.pallas import tpu as pltpu
from jax.experimental.pallas import tpu_sc as plsc

BATCH = 16384
HIST_LEN = 50
EMBEDDING_DIM = 64
TOTAL = BATCH * HIST_LEN

_INFO = plsc.get_sparse_core_info()
NUM_CORES = _INFO.num_cores
NUM_SUBCORES = _INFO.num_subcores
NUM_WORKERS = NUM_CORES * NUM_SUBCORES

BATCH_PER_WORKER = BATCH // NUM_WORKERS
PER_WORKER = BATCH_PER_WORKER * HIST_LEN
GB = 8
CHUNK = GB * HIST_LEN
STEPS = BATCH_PER_WORKER // GB
NBUF = 2


def _gather_kernel(table_hbm, idx_hbm, out_hbm, idx_all, rows, *sems):
    gsems = sems[:NBUF]
    ssems = sems[NBUF:]
    wid = lax.axis_index("s") * NUM_CORES + lax.axis_index("c")
    row_base = wid * PER_WORKER
    batch_base = wid * BATCH_PER_WORKER

    def idx_slice(i):
        return idx_all.at[pl.ds(i * CHUNK, CHUNK)]

    def start_gather(i, b):
        pltpu.async_copy(table_hbm.at[idx_slice(i)], rows.at[b], gsems[b])

    def wait_gather(i, b):
        pltpu.make_async_copy(
            table_hbm.at[idx_slice(i)], rows.at[b], gsems[b]
        ).wait()

    def start_store(i, b):
        for g in range(GB):
            pltpu.async_copy(
                rows.at[b, pl.ds(g * HIST_LEN, HIST_LEN)],
                out_hbm.at[batch_base + i * GB + g],
                ssems[b],
            )

    def wait_store(i, b):
        for g in range(GB):
            pltpu.make_async_copy(
                rows.at[b, pl.ds(g * HIST_LEN, HIST_LEN)],
                out_hbm.at[batch_base + i * GB + g],
                ssems[b],
            ).wait()

    pltpu.sync_copy(idx_hbm.at[pl.ds(row_base, PER_WORKER)], idx_all)

    for b in range(NBUF):
        start_gather(b, b)

    @pl.loop(0, STEPS, step=NBUF)
    def _outer(g):
        for b in range(NBUF):
            i = g + b
            wait_gather(i, b)
            start_store(i, b)

            @pl.when(i + NBUF < STEPS)
            def _():
                wait_store(i, b)
                start_gather(i + NBUF, b)

    for b in range(NBUF):
        wait_store(STEPS - NBUF + b, b)


@jax.jit
def _gather(table, idx_flat):
    mesh = plsc.VectorSubcoreMesh(core_axis_name="c", subcore_axis_name="s")
    run = functools.partial(
        pl.kernel,
        mesh=mesh,
        out_type=jax.ShapeDtypeStruct(
            (BATCH, HIST_LEN, EMBEDDING_DIM), jnp.float32
        ),
        scratch_types=[
            pltpu.VMEM((PER_WORKER,), jnp.int32),
            pltpu.VMEM((NBUF, CHUNK, EMBEDDING_DIM), jnp.float32),
        ]
        + [pltpu.SemaphoreType.DMA] * (2 * NBUF),
        compiler_params=pltpu.CompilerParams(use_tc_tiling_on_sc=False),
    )(_gather_kernel)
    return run(table, idx_flat)


def kernel(token_ids, embedding_matrix):
    idx_flat = token_ids.reshape(TOTAL).astype(jnp.int32)
    return _gather(embedding_matrix, idx_flat)
